# baseline (speedup 1.0000x reference)
import functools

import jax
import jax.numpy as jnp
from jax import lax
from jax.experimental import pallas as pl
from jax.experimental.pallas import tpu as pltpu
from jax.experimental.pallas import tpu_sc as plsc

B = 16384
D = 64
L = 16
NC = 2
NS = 16
NW = NC * NS
BPW = B // NW
CH = 128
NCH = BPW // CH

_mesh = plsc.VectorSubcoreMesh(core_axis_name="c", subcore_axis_name="s")


@functools.partial(
    pl.kernel,
    mesh=_mesh,
    compiler_params=pltpu.CompilerParams(
        needs_layout_passes=False, use_tc_tiling_on_sc=False
    ),
    out_type=jax.ShapeDtypeStruct((B,), jnp.float32),
    scratch_types=[
        pltpu.VMEM((NCH, CH), jnp.int32),
        pltpu.VMEM((NCH, CH), jnp.int32),
        pltpu.VMEM((NCH, CH), jnp.int32),
        pltpu.VMEM((BPW, D), jnp.float32),
        pltpu.VMEM((BPW, D), jnp.float32),
        pltpu.VMEM((BPW, D), jnp.float32),
        pltpu.VMEM((BPW,), jnp.float32),
        pltpu.SemaphoreType.DMA,
    ],
)
def _transe_sc(hidx_hbm, ridx_hbm, tidx_hbm, ent_hbm, rel_hbm, out_hbm,
               hidx_v, ridx_v, tidx_v, h_v, r_v, t_v, out_v, sem):
    wid = lax.axis_index("s") * NC + lax.axis_index("c")
    base = wid * BPW

    for j in range(NCH):
        pltpu.sync_copy(hidx_hbm.at[pl.ds(base + j * CH, CH)], hidx_v.at[j])
        pltpu.sync_copy(ridx_hbm.at[pl.ds(base + j * CH, CH)], ridx_v.at[j])
        pltpu.sync_copy(tidx_hbm.at[pl.ds(base + j * CH, CH)], tidx_v.at[j])

    copies = []
    for j in range(NCH):
        dst = pl.ds(j * CH, CH)
        copies.append(pltpu.async_copy(ent_hbm.at[hidx_v.at[j]], h_v.at[dst], sem))
        copies.append(pltpu.async_copy(rel_hbm.at[ridx_v.at[j]], r_v.at[dst], sem))
        copies.append(pltpu.async_copy(ent_hbm.at[tidx_v.at[j]], t_v.at[dst], sem))
    for c in copies:
        c.wait()

    lane = lax.iota(jnp.int32, L)

    def group_body(g, carry):
        rows = g * L + lane

        def d_body(d, acc):
            cols = jnp.full((L,), d, jnp.int32)
            hv = plsc.load_gather(h_v, [rows, cols])
            rv = plsc.load_gather(r_v, [rows, cols])
            tv = plsc.load_gather(t_v, [rows, cols])
            return acc + jnp.abs(hv + rv - tv)

        acc = lax.fori_loop(0, D, d_body, jnp.zeros((L,), jnp.float32))
        out_v[pl.ds(g * L, L)] = acc
        return carry

    lax.fori_loop(0, BPW // L, group_body, 0)
    pltpu.sync_copy(out_v, out_hbm.at[pl.ds(base, BPW)])


def kernel(triples, ent, rel):
    tr = triples.astype(jnp.int32)
    h_idx = tr[:, 0]
    r_idx = tr[:, 1]
    t_idx = tr[:, 2]
    return _transe_sc(h_idx, r_idx, t_idx, ent, rel)
